# baseline (speedup 1.0000x reference)
import functools

import jax
import jax.numpy as jnp
from jax import lax
from jax.experimental import pallas as pl
from jax.experimental.pallas import tpu as pltpu
from jax.experimental.pallas import tpu_sc as plsc

VOCAB = 1_000_000
EMBED_DIM = 32
BATCH = 4096
SEQ_LEN = 200

_NC = 2
_NS = 16
_NW = _NC * _NS
_NB = BATCH // 128


_NVB = VOCAB // 128
_NVB_MAIN = (_NVB // _NW) * _NW
_TAIL = VOCAB - _NVB * 128


def _make_sc_detile():
  mesh = plsc.VectorSubcoreMesh(core_axis_name="c", subcore_axis_name="s")

  @functools.partial(
      pl.kernel,
      out_type=jax.ShapeDtypeStruct((VOCAB * EMBED_DIM,), jnp.float32),
      mesh=mesh,
      compiler_params=pltpu.CompilerParams(
          use_tc_tiling_on_sc=True, needs_layout_passes=False),
      scratch_types=[
          pltpu.VMEM((8, 128), jnp.float32),
          pltpu.VMEM((8, 128), jnp.float32),
          pltpu.VMEM((8, 128), jnp.float32),
          pltpu.VMEM((8, 128), jnp.float32),
          pltpu.VMEM((8, 128), jnp.float32),
          pltpu.VMEM((8, 128), jnp.float32),
          pltpu.VMEM((8, 128), jnp.float32),
          pltpu.VMEM((8, 128), jnp.float32),
          pltpu.VMEM((128 * 33,), jnp.float32),
          pltpu.VMEM((128 * 33,), jnp.float32),
          pltpu.VMEM((128 * EMBED_DIM,), jnp.float32),
          pltpu.VMEM((128 * EMBED_DIM,), jnp.float32),
          pltpu.SemaphoreType.DMA,
          pltpu.SemaphoreType.DMA,
          pltpu.SemaphoreType.DMA,
          pltpu.SemaphoreType.DMA,
      ],
  )
  def sc_detile(wt_hbm, wtail_hbm, wf_hbm,
                sb0, sb1, sb2, sb3, sb4, sb5, sb6, sb7,
                pb0, pb1, db0, db1, g0, g1, s0, s1):
    wid = lax.axis_index("s") * _NC + lax.axis_index("c")
    sbufs = (sb0, sb1, sb2, sb3, sb4, sb5, sb6, sb7)
    pbufs = (pb0, pb1)
    dbufs = (db0, db1)
    gsems = (g0, g1)
    ssems = (s0, s1)
    iota = lax.iota(jnp.int32, 16)
    iota33 = iota * 33

    def start_in(blk, b):
      for i in range(4):
        pltpu.async_copy(
            wt_hbm.at[pl.ds(i * 8, 8), pl.ds(blk * 128, 128)],
            sbufs[b * 4 + i], gsems[b])

    def wait_in(b):
      for i in range(4):
        pltpu.make_async_copy(
            wt_hbm.at[pl.ds(0, 8), pl.ds(0, 128)],
            sbufs[b * 4 + i], gsems[b]).wait()

    def transpose(b):
      for v0 in range(8):
        vals = [sbufs[b * 4 + i][dsub, pl.ds(v0 * 16, 16)]
                for i in range(4) for dsub in range(8)]
        idxs = [iota33 + (v0 * 528 + d) for d in range(32)]
        for d in range(32):
          plsc.store_scatter(pbufs[b], [idxs[d]], vals[d])
        for v in range(v0 * 16, v0 * 16 + 16):
          dbufs[b][pl.ds(v * 32, 16)] = pbufs[b][pl.ds(v * 33, 16)]
          dbufs[b][pl.ds(v * 32 + 16, 16)] = pbufs[b][pl.ds(v * 33 + 16, 16)]

    def start_out(blk, b):
      pltpu.async_copy(
          dbufs[b], wf_hbm.at[pl.ds(blk * 4096, 4096)], ssems[b])

    def wait_out(b):
      pltpu.make_async_copy(
          dbufs[b], wf_hbm.at[pl.ds(0, 4096)], ssems[b]).wait()

    start_in(wid, 0)
    start_in(wid + _NW, 1)

    @pl.loop(0, _NVB_MAIN // _NW, step=2)
    def _(t):
      for b in range(2):
        tb = t + b
        blk = wid + tb * _NW
        wait_in(b)

        @pl.when(tb >= 2)
        def _():
          wait_out(b)

        transpose(b)
        start_out(blk, b)

        @pl.when(tb + 2 < _NVB_MAIN // _NW)
        def _():
          start_in(blk + 2 * _NW, b)

    for b in range(2):
      wait_out(b)

    @pl.when(wid < _NVB - _NVB_MAIN)
    def _():
      blk = _NVB_MAIN + wid
      start_in(blk, 0)
      wait_in(0)
      transpose(0)
      start_out(blk, 0)
      wait_out(0)

    @pl.when(wid == 4)
    def _():
      pltpu.sync_copy(wtail_hbm, wf_hbm.at[pl.ds(_NVB * 4096, _TAIL * EMBED_DIM)])

  return sc_detile


def _make_sc_gather():
  mesh = plsc.VectorSubcoreMesh(core_axis_name="c", subcore_axis_name="s")

  @functools.partial(
      pl.kernel,
      out_type=jax.ShapeDtypeStruct((SEQ_LEN, 4, _NB, 8, 128), jnp.float32),
      mesh=mesh,
      compiler_params=pltpu.CompilerParams(
          use_tc_tiling_on_sc=False, needs_layout_passes=False),
      scratch_types=[
          pltpu.VMEM((SEQ_LEN, 128), jnp.int32),
          pltpu.VMEM((2, 128, EMBED_DIM), jnp.float32),
          pltpu.VMEM((2, EMBED_DIM, 129), jnp.float32),
          pltpu.SemaphoreType.DMA,
          pltpu.SemaphoreType.DMA,
          pltpu.SemaphoreType.DMA,
          pltpu.SemaphoreType.DMA,
      ],
  )
  def sc_gather(x_hbm, w_hbm, out_hbm, idx_v, gbuf_v, tbuf_v, g0, g1, s0, s1):
    wid = lax.axis_index("s") * _NC + lax.axis_index("c")
    pltpu.sync_copy(x_hbm.at[:, pl.ds(wid * 128, 128)], idx_v)

    gsems = (g0, g1)
    ssems = (s0, s1)
    iota = lax.iota(jnp.int32, 16)
    d_idx = (iota, iota + 16)

    def store_tile(b, jb):
      for dblk in range(4):
        pltpu.async_copy(
            tbuf_v.at[b, pl.ds(dblk * 8, 8), pl.ds(0, 128)],
            out_hbm.at[jb, dblk, wid], ssems[b])

    def wait_store(b):
      for dblk in range(4):
        pltpu.make_async_copy(
            tbuf_v.at[b, pl.ds(dblk * 8, 8), pl.ds(0, 128)],
            out_hbm.at[0, dblk, wid], ssems[b]).wait()

    pltpu.async_copy(w_hbm.at[idx_v.at[0]], gbuf_v.at[0], g0)
    pltpu.async_copy(w_hbm.at[idx_v.at[1]], gbuf_v.at[1], g1)

    @pl.loop(0, SEQ_LEN, step=2)
    def _(j):
      for b in range(2):
        jb = j + b
        pltpu.make_async_copy(
            w_hbm.at[idx_v.at[0]], gbuf_v.at[b], gsems[b]).wait()

        @pl.when(jb >= 2)
        def _():
          wait_store(b)

        for b0 in range(0, 128, 8):
          vals = [
              (gbuf_v[b, b0 + r, pl.ds(0, 16)], gbuf_v[b, b0 + r, pl.ds(16, 16)])
              for r in range(8)
          ]
          bcols = [jnp.full((16,), b0 + r, jnp.int32) for r in range(8)]
          for r in range(8):
            for h in range(2):
              plsc.store_scatter(
                  tbuf_v.at[b], [d_idx[h], bcols[r]], vals[r][h])

        store_tile(b, jb)

        @pl.when(jb + 2 < SEQ_LEN)
        def _():
          pltpu.async_copy(
              w_hbm.at[idx_v.at[jb + 2]], gbuf_v.at[b], gsems[b])

    for b in range(2):
      wait_store(b)

  return sc_gather


_sc_detile = _make_sc_detile()
_sc_gather = _make_sc_gather()


@jax.jit
def kernel(x, w):
  xt = jnp.swapaxes(x, 0, 1).astype(jnp.int32)
  wtail = jax.lax.slice(w, (_NVB * 128, 0), (VOCAB, EMBED_DIM))
  w_rm = _sc_detile(
      jnp.swapaxes(w, 0, 1),
      wtail.reshape(_TAIL * EMBED_DIM)).reshape(VOCAB, EMBED_DIM)
  out5 = _sc_gather(xt, w_rm)
  return out5.transpose(2, 4, 0, 1, 3).reshape(BATCH, SEQ_LEN, EMBED_DIM)


# speedup vs baseline: 1.1278x; 1.1278x over previous
import functools

import jax
import jax.numpy as jnp
from jax import lax
from jax.experimental import pallas as pl
from jax.experimental.pallas import tpu as pltpu
from jax.experimental.pallas import tpu_sc as plsc

VOCAB = 1_000_000
EMBED_DIM = 32
BATCH = 4096
SEQ_LEN = 200

_NC = 2
_NS = 16
_NW = _NC * _NS
_NB = BATCH // 128


_NVB = VOCAB // 128
_NVB_MAIN = (_NVB // _NW) * _NW
_TAIL = VOCAB - _NVB * 128


def _make_sc_detile():
  mesh = plsc.VectorSubcoreMesh(core_axis_name="c", subcore_axis_name="s")

  @functools.partial(
      pl.kernel,
      out_type=jax.ShapeDtypeStruct((VOCAB * EMBED_DIM,), jnp.float32),
      mesh=mesh,
      compiler_params=pltpu.CompilerParams(
          use_tc_tiling_on_sc=True, needs_layout_passes=False),
      scratch_types=[
          pltpu.VMEM((8, 128), jnp.float32),
          pltpu.VMEM((8, 128), jnp.float32),
          pltpu.VMEM((8, 128), jnp.float32),
          pltpu.VMEM((8, 128), jnp.float32),
          pltpu.VMEM((8, 128), jnp.float32),
          pltpu.VMEM((8, 128), jnp.float32),
          pltpu.VMEM((8, 128), jnp.float32),
          pltpu.VMEM((8, 128), jnp.float32),
          pltpu.VMEM((128 * 33,), jnp.float32),
          pltpu.VMEM((128 * 33,), jnp.float32),
          pltpu.VMEM((128 * EMBED_DIM,), jnp.float32),
          pltpu.VMEM((128 * EMBED_DIM,), jnp.float32),
          pltpu.SemaphoreType.DMA,
          pltpu.SemaphoreType.DMA,
          pltpu.SemaphoreType.DMA,
          pltpu.SemaphoreType.DMA,
      ],
  )
  def sc_detile(wt_hbm, wtail_hbm, wf_hbm,
                sb0, sb1, sb2, sb3, sb4, sb5, sb6, sb7,
                pb0, pb1, db0, db1, g0, g1, s0, s1):
    wid = lax.axis_index("s") * _NC + lax.axis_index("c")
    sbufs = (sb0, sb1, sb2, sb3, sb4, sb5, sb6, sb7)
    pbufs = (pb0, pb1)
    dbufs = (db0, db1)
    gsems = (g0, g1)
    ssems = (s0, s1)
    iota = lax.iota(jnp.int32, 16)
    iota33 = iota * 33

    def start_in(blk, b):
      for i in range(4):
        pltpu.async_copy(
            wt_hbm.at[pl.ds(i * 8, 8), pl.ds(blk * 128, 128)],
            sbufs[b * 4 + i], gsems[b])

    def wait_in(b):
      for i in range(4):
        pltpu.make_async_copy(
            wt_hbm.at[pl.ds(0, 8), pl.ds(0, 128)],
            sbufs[b * 4 + i], gsems[b]).wait()

    def transpose(b):
      for i in range(4):
        for dsub in range(8):
          d = i * 8 + dsub
          vals = [sbufs[b * 4 + i][dsub, pl.ds(v0 * 16, 16)]
                  for v0 in range(8)]
          idxs = [iota33 + (v0 * 528 + d) for v0 in range(8)]
          for v0 in range(8):
            plsc.store_scatter(pbufs[b], [idxs[v0]], vals[v0])
      for v in range(128):
        dbufs[b][pl.ds(v * 32, 16)] = pbufs[b][pl.ds(v * 33, 16)]
        dbufs[b][pl.ds(v * 32 + 16, 16)] = pbufs[b][pl.ds(v * 33 + 16, 16)]

    def start_out(blk, b):
      pltpu.async_copy(
          dbufs[b], wf_hbm.at[pl.ds(blk * 4096, 4096)], ssems[b])

    def wait_out(b):
      pltpu.make_async_copy(
          dbufs[b], wf_hbm.at[pl.ds(0, 4096)], ssems[b]).wait()

    start_in(wid, 0)
    start_in(wid + _NW, 1)

    @pl.loop(0, _NVB_MAIN // _NW, step=2)
    def _(t):
      for b in range(2):
        tb = t + b
        blk = wid + tb * _NW
        wait_in(b)

        @pl.when(tb >= 2)
        def _():
          wait_out(b)

        transpose(b)
        start_out(blk, b)

        @pl.when(tb + 2 < _NVB_MAIN // _NW)
        def _():
          start_in(blk + 2 * _NW, b)

    for b in range(2):
      wait_out(b)

    @pl.when(wid < _NVB - _NVB_MAIN)
    def _():
      blk = _NVB_MAIN + wid
      start_in(blk, 0)
      wait_in(0)
      transpose(0)
      start_out(blk, 0)
      wait_out(0)

    @pl.when(wid == 4)
    def _():
      pltpu.sync_copy(wtail_hbm, wf_hbm.at[pl.ds(_NVB * 4096, _TAIL * EMBED_DIM)])

  return sc_detile


def _make_sc_gather():
  mesh = plsc.VectorSubcoreMesh(core_axis_name="c", subcore_axis_name="s")

  @functools.partial(
      pl.kernel,
      out_type=jax.ShapeDtypeStruct((SEQ_LEN, 4, _NB, 8, 128), jnp.float32),
      mesh=mesh,
      compiler_params=pltpu.CompilerParams(
          use_tc_tiling_on_sc=False, needs_layout_passes=False),
      scratch_types=[
          pltpu.VMEM((SEQ_LEN, 128), jnp.int32),
          pltpu.VMEM((2, 128, EMBED_DIM), jnp.float32),
          pltpu.VMEM((2, EMBED_DIM, 129), jnp.float32),
          pltpu.SemaphoreType.DMA,
          pltpu.SemaphoreType.DMA,
          pltpu.SemaphoreType.DMA,
          pltpu.SemaphoreType.DMA,
      ],
  )
  def sc_gather(x_hbm, w_hbm, out_hbm, idx_v, gbuf_v, tbuf_v, g0, g1, s0, s1):
    wid = lax.axis_index("s") * _NC + lax.axis_index("c")
    pltpu.sync_copy(x_hbm.at[:, pl.ds(wid * 128, 128)], idx_v)

    gsems = (g0, g1)
    ssems = (s0, s1)
    iota = lax.iota(jnp.int32, 16)
    d_idx = (iota, iota + 16)

    def store_tile(b, jb):
      for dblk in range(4):
        pltpu.async_copy(
            tbuf_v.at[b, pl.ds(dblk * 8, 8), pl.ds(0, 128)],
            out_hbm.at[jb, dblk, wid], ssems[b])

    def wait_store(b):
      for dblk in range(4):
        pltpu.make_async_copy(
            tbuf_v.at[b, pl.ds(dblk * 8, 8), pl.ds(0, 128)],
            out_hbm.at[0, dblk, wid], ssems[b]).wait()

    pltpu.async_copy(w_hbm.at[idx_v.at[0]], gbuf_v.at[0], g0)
    pltpu.async_copy(w_hbm.at[idx_v.at[1]], gbuf_v.at[1], g1)

    @pl.loop(0, SEQ_LEN, step=2)
    def _(j):
      for b in range(2):
        jb = j + b
        pltpu.make_async_copy(
            w_hbm.at[idx_v.at[0]], gbuf_v.at[b], gsems[b]).wait()

        @pl.when(jb >= 2)
        def _():
          wait_store(b)

        for b0 in range(0, 128, 8):
          vals = [
              (gbuf_v[b, b0 + r, pl.ds(0, 16)], gbuf_v[b, b0 + r, pl.ds(16, 16)])
              for r in range(8)
          ]
          bcols = [jnp.full((16,), b0 + r, jnp.int32) for r in range(8)]
          for r in range(8):
            for h in range(2):
              plsc.store_scatter(
                  tbuf_v.at[b], [d_idx[h], bcols[r]], vals[r][h])

        store_tile(b, jb)

        @pl.when(jb + 2 < SEQ_LEN)
        def _():
          pltpu.async_copy(
              w_hbm.at[idx_v.at[jb + 2]], gbuf_v.at[b], gsems[b])

    for b in range(2):
      wait_store(b)

  return sc_gather


_sc_detile = _make_sc_detile()
_sc_gather = _make_sc_gather()


@jax.jit
def kernel(x, w):
  xt = jnp.swapaxes(x, 0, 1).astype(jnp.int32)
  wtail = jax.lax.slice(w, (_NVB * 128, 0), (VOCAB, EMBED_DIM))
  w_rm = _sc_detile(
      jnp.swapaxes(w, 0, 1),
      wtail.reshape(_TAIL * EMBED_DIM)).reshape(VOCAB, EMBED_DIM)
  out5 = _sc_gather(xt, w_rm)
  return out5.transpose(2, 4, 0, 1, 3).reshape(BATCH, SEQ_LEN, EMBED_DIM)
